# Initial kernel scaffold; baseline (speedup 1.0000x reference)
import jax
import jax.numpy as jnp
from jax.experimental import pallas as pl


def kernel(x, edge_index, W_ih, W_hh, b_lstm, k_param, x_param, W_self0, W_neigh0, b0, W_self1, W_neigh1, b1, W_head, b_head):
    raise NotImplementedError("write your pallas kernel here")


# speedup vs baseline: 5.2404x; 5.2404x over previous
import jax
import jax.numpy as jnp
from jax import lax
from jax.experimental import pallas as pl
from jax.experimental.pallas import tpu as pltpu
from jax.experimental.pallas import tpu_sc as plsc

N = 10000
E = 320000
D = 128
H = 128
T = 12
HOR = 24
DT = 1.0

NC = 2
NS = 16
NW = NC * NS
EPW = E // NW
K = 80
NCHUNK = EPW // K
CB = 25
SB = NCHUNK // CB
APAD = 10240
RPS = APAD // NS
ZROWS = 128
LN = 16

BN = 1000
GRID = N // BN

EW_R = E // 128


def _sc_body(h_hbm, src_hbm, dst_hbm, w_hbm, out_hbm,
             agg_sh, src_t, dst_t, w_t, rows, sem):
    cid = lax.axis_index("c")
    sid = lax.axis_index("s")
    wid = sid * NC + cid

    zv = jnp.zeros((LN,), jnp.float32)

    def zrow(i, carry):
        for q in range(H // LN):
            rows[i, pl.ds(q * LN, LN)] = zv
        return carry

    lax.fori_loop(0, K, zrow, 0)
    for r in range(RPS // K):
        pltpu.sync_copy(rows, agg_sh.at[pl.ds(sid * RPS + r * K, K)])

    plsc.subcore_barrier()

    def superblock(s, carry):
        pltpu.sync_copy(src_hbm.at[wid, s], src_t)
        pltpu.sync_copy(dst_hbm.at[wid, s], dst_t)
        pltpu.sync_copy(w_hbm.at[wid, s], w_t)

        def chunk(j, c1):
            pltpu.async_copy(h_hbm.at[src_t.at[j]], rows, sem).wait()

            def group(g, c2):
                wv = w_t[j, pl.ds(g * LN, LN)]
                base = g * LN
                for l in range(LN):
                    ws = wv[l]
                    for q in range(H // LN):
                        rows[base + l, pl.ds(q * LN, LN)] = (
                            rows[base + l, pl.ds(q * LN, LN)] * ws)
                return c2

            lax.fori_loop(0, K // LN, group, 0)
            pltpu.sync_copy(rows, agg_sh.at[dst_t.at[j]], add=True)
            return c1

        lax.fori_loop(0, CB, chunk, 0)
        return carry

    lax.fori_loop(0, SB, superblock, 0)
    plsc.subcore_barrier()

    @pl.when(sid < NS - 1)
    def _full():
        pltpu.sync_copy(agg_sh.at[pl.ds(sid * RPS, RPS)],
                        out_hbm.at[cid, pl.ds(sid * RPS, RPS)])

    @pl.when(sid == NS - 1)
    def _tail():
        pltpu.sync_copy(agg_sh.at[pl.ds((NS - 1) * RPS, N - (NS - 1) * RPS)],
                        out_hbm.at[cid, pl.ds((NS - 1) * RPS,
                                              N - (NS - 1) * RPS)])


_SC_KERNEL_CACHE = []


def _sc_gather_scatter(h, src, dst, w):
    if not _SC_KERNEL_CACHE:
        _SC_KERNEL_CACHE.append(pl.kernel(
            _sc_body,
            out_type=jax.ShapeDtypeStruct((NC, N, H), jnp.float32),
            mesh=plsc.VectorSubcoreMesh(core_axis_name="c",
                                        subcore_axis_name="s",
                                        num_cores=NC, num_subcores=NS),
            scratch_types=[
                pltpu.VMEM_SHARED((APAD, H), jnp.float32),
                pltpu.VMEM((CB, K), jnp.int32),
                pltpu.VMEM((CB, K), jnp.int32),
                pltpu.VMEM((CB, K), jnp.float32),
                pltpu.VMEM((K, H), jnp.float32),
                pltpu.SemaphoreType.DMA,
            ],
        ))
    return _SC_KERNEL_CACHE[0](h, src, dst, w)


def _lstm_body(x_ref, wih_ref, whh_ref, b_ref, h_ref):
    wih = wih_ref[...]
    whh = whh_ref[...]
    b = b_ref[...]
    h = jnp.zeros((BN, H), jnp.float32)
    c = jnp.zeros((BN, H), jnp.float32)
    for t in range(T):
        g = (jnp.dot(x_ref[:, t, :], wih, preferred_element_type=jnp.float32)
             + jnp.dot(h, whh, preferred_element_type=jnp.float32) + b)
        i = jax.nn.sigmoid(g[:, :H])
        f = jax.nn.sigmoid(g[:, H:2 * H])
        gg = jnp.tanh(g[:, 2 * H:3 * H])
        o = jax.nn.sigmoid(g[:, 3 * H:])
        c = f * c + i * gg
        h = o * jnp.tanh(c)
    h_ref[...] = h


def _lstm_call(xs, W_ih, W_hh, b2):
    return pl.pallas_call(
        _lstm_body,
        grid=(GRID,),
        in_specs=[
            pl.BlockSpec((BN, T, D), lambda i: (i, 0, 0)),
            pl.BlockSpec((D, 4 * H), lambda i: (0, 0)),
            pl.BlockSpec((H, 4 * H), lambda i: (0, 0)),
            pl.BlockSpec((1, 4 * H), lambda i: (0, 0)),
        ],
        out_specs=pl.BlockSpec((BN, H), lambda i: (i, 0)),
        out_shape=jax.ShapeDtypeStruct((N, H), jnp.float32),
    )(xs, W_ih, W_hh, b2)


def _edgew_body(k_ref, x_ref, o_ref):
    k = jax.nn.softplus(k_ref[...]) + 1e-3
    xr = 0.5 * jax.nn.sigmoid(x_ref[...])
    denom = k * (1.0 - xr) + 0.5 * DT
    o_ref[...] = (0.5 * DT - k * xr) / denom


def _edgew_call(kp, xp):
    return pl.pallas_call(
        _edgew_body,
        out_shape=jax.ShapeDtypeStruct((EW_R, 128), jnp.float32),
    )(kp, xp)


def _glayer_body(h_ref, p_ref, ws_ref, wn_ref, b_ref, o_ref):
    agg = p_ref[0] + p_ref[1]
    o_ref[...] = jnp.maximum(
        jnp.dot(h_ref[...], ws_ref[...], preferred_element_type=jnp.float32)
        + jnp.dot(agg, wn_ref[...], preferred_element_type=jnp.float32)
        + b_ref[...], 0.0)


def _glayer_call(h, p, Ws, Wn, b2):
    return pl.pallas_call(
        _glayer_body,
        grid=(GRID,),
        in_specs=[
            pl.BlockSpec((BN, H), lambda i: (i, 0)),
            pl.BlockSpec((NC, BN, H), lambda i: (0, i, 0)),
            pl.BlockSpec((H, H), lambda i: (0, 0)),
            pl.BlockSpec((H, H), lambda i: (0, 0)),
            pl.BlockSpec((1, H), lambda i: (0, 0)),
        ],
        out_specs=pl.BlockSpec((BN, H), lambda i: (i, 0)),
        out_shape=jax.ShapeDtypeStruct((N, H), jnp.float32),
    )(h, p, Ws, Wn, b2)


def _glayer_head_body(h_ref, p_ref, ws_ref, wn_ref, b_ref, wh_ref, bh_ref,
                      o_ref):
    agg = p_ref[0] + p_ref[1]
    h2 = jnp.maximum(
        jnp.dot(h_ref[...], ws_ref[...], preferred_element_type=jnp.float32)
        + jnp.dot(agg, wn_ref[...], preferred_element_type=jnp.float32)
        + b_ref[...], 0.0)
    o_ref[...] = (jnp.dot(h2, wh_ref[...], preferred_element_type=jnp.float32)
                  + bh_ref[...])


def _glayer_head_call(h, p, Ws, Wn, b2, Whp, bhp):
    return pl.pallas_call(
        _glayer_head_body,
        grid=(GRID,),
        in_specs=[
            pl.BlockSpec((BN, H), lambda i: (i, 0)),
            pl.BlockSpec((NC, BN, H), lambda i: (0, i, 0)),
            pl.BlockSpec((H, H), lambda i: (0, 0)),
            pl.BlockSpec((H, H), lambda i: (0, 0)),
            pl.BlockSpec((1, H), lambda i: (0, 0)),
            pl.BlockSpec((H, 128), lambda i: (0, 0)),
            pl.BlockSpec((1, 128), lambda i: (0, 0)),
        ],
        out_specs=pl.BlockSpec((BN, 128), lambda i: (i, 0)),
        out_shape=jax.ShapeDtypeStruct((N, 128), jnp.float32),
    )(h, p, Ws, Wn, b2, Whp, bhp)


def kernel(x, edge_index, W_ih, W_hh, b_lstm, k_param, x_param,
           W_self0, W_neigh0, b0, W_self1, W_neigh1, b1, W_head, b_head):
    xs = x[0]
    src = edge_index[0].reshape(NW, SB, CB, K)
    dst = edge_index[1].reshape(NW, SB, CB, K)
    w = _edgew_call(k_param.reshape(EW_R, 128),
                    x_param.reshape(EW_R, 128)).reshape(NW, SB, CB, K)

    h = _lstm_call(xs, W_ih, W_hh, b_lstm.reshape(1, 4 * H))

    p0 = _sc_gather_scatter(h, src, dst, w)
    h1 = _glayer_call(h, p0, W_self0, W_neigh0, b0.reshape(1, H))

    p1 = _sc_gather_scatter(h1, src, dst, w)
    Whp = jnp.zeros((H, 128), jnp.float32).at[:, :HOR].set(W_head)
    bhp = jnp.zeros((1, 128), jnp.float32).at[0, :HOR].set(b_head)
    out = _glayer_head_call(h1, p1, W_self1, W_neigh1, b1.reshape(1, H),
                            Whp, bhp)
    return out[:, :HOR][None]


# speedup vs baseline: 7.6603x; 1.4618x over previous
import jax
import jax.numpy as jnp
from jax import lax
from jax.experimental import pallas as pl
from jax.experimental.pallas import tpu as pltpu
from jax.experimental.pallas import tpu_sc as plsc

N = 10000
E = 320000
D = 128
H = 128
T = 12
HOR = 24
DT = 1.0

NC = 2
NS = 16
NW = NC * NS
EPW = E // NW
K = 80
NCHUNK = EPW // K
APAD = 10240
RPS = APAD // NS
ZROWS = 128
LN = 16

BN = 1000
GRID = N // BN

EW_R = E // 128


def _sc_body(h_hbm, src_hbm, dst_hbm, w_hbm, out_hbm, agg_sh,
             src0, src1, dst0, dst1, dsc0, dsc1, w0, w1, rows0, rows1,
             gsem0, gsem1, isem0, isem1, ssem0, ssem1):
    cid = lax.axis_index("c")
    sid = lax.axis_index("s")
    wid = sid * NC + cid

    srcb = (src0, src1)
    dstb = (dst0, dst1)
    dscb = (dsc0, dsc1)
    wb = (w0, w1)
    rowsb = (rows0, rows1)
    gsem = (gsem0, gsem1)
    isem = (isem0, isem1)
    ssem = (ssem0, ssem1)

    def stage(j, s):
        pltpu.async_copy(src_hbm.at[wid, j], srcb[s], isem[s])
        pltpu.async_copy(dst_hbm.at[wid, j], dstb[s], isem[s])
        pltpu.async_copy(w_hbm.at[wid, j], wb[s], isem[s])

    def wait_stage(s):
        pltpu.make_async_copy(src_hbm.at[0, 0], srcb[s], isem[s]).wait()
        pltpu.make_async_copy(dst_hbm.at[0, 0], dstb[s], isem[s]).wait()
        pltpu.make_async_copy(w_hbm.at[0, 0], wb[s], isem[s]).wait()

    def gissue(s):
        pltpu.async_copy(h_hbm.at[srcb[s].at[0]], rowsb[s], gsem[s])

    def gwait(s):
        pltpu.make_async_copy(h_hbm.at[srcb[s].at[0]], rowsb[s],
                              gsem[s]).wait()

    def scatter_issue(s):
        pltpu.async_copy(rowsb[s], agg_sh.at[dscb[s].at[0]], ssem[s],
                         add=True)

    def wait_scatter(s):
        pltpu.make_async_copy(rowsb[s], agg_sh.at[dscb[s].at[0]],
                              ssem[s]).wait()

    def process(s):
        def group(g, c2):
            wv = wb[s][0, pl.ds(g * LN, LN)]
            base = g * LN
            for l in range(LN):
                ws = wv[l]
                for q in range(H // LN):
                    rowsb[s][base + l, pl.ds(q * LN, LN)] = (
                        rowsb[s][base + l, pl.ds(q * LN, LN)] * ws)
            return c2

        lax.fori_loop(0, K // LN, group, 0)
        for g in range(K // LN):
            dscb[s][0, pl.ds(g * LN, LN)] = dstb[s][0, pl.ds(g * LN, LN)]
        scatter_issue(s)

    stage(0, 0)
    stage(1, 1)
    zv = jnp.zeros((LN,), jnp.float32)

    def zrow(i, carry):
        for q in range(H // LN):
            rows1[i, pl.ds(q * LN, LN)] = zv
        return carry

    lax.fori_loop(0, K, zrow, 0)
    for r in range(RPS // K):
        pltpu.sync_copy(rows1, agg_sh.at[pl.ds(sid * RPS + r * K, K)])
    wait_stage(0)
    gissue(0)
    plsc.subcore_barrier()

    def half(j, s):
        @pl.when((j + 1 < NCHUNK) & (j >= 1))
        def _drain():
            wait_scatter(1 - s)

        @pl.when(j + 1 < NCHUNK)
        def _prefetch():
            wait_stage(1 - s)
            gissue(1 - s)

        @pl.when(j < NCHUNK)
        def _proc():
            gwait(s)
            process(s)

        @pl.when(j + 2 < NCHUNK)
        def _restage():
            stage(j + 2, s)

    def pair(jj, carry):
        half(2 * jj, 0)
        half(2 * jj + 1, 1)
        return carry

    lax.fori_loop(0, (NCHUNK + 1) // 2, pair, 0)
    wait_scatter(0)
    wait_scatter(1)
    plsc.subcore_barrier()

    @pl.when(sid < NS - 1)
    def _full():
        pltpu.sync_copy(agg_sh.at[pl.ds(sid * RPS, RPS)],
                        out_hbm.at[cid, pl.ds(sid * RPS, RPS)])

    @pl.when(sid == NS - 1)
    def _tail():
        pltpu.sync_copy(agg_sh.at[pl.ds((NS - 1) * RPS, N - (NS - 1) * RPS)],
                        out_hbm.at[cid, pl.ds((NS - 1) * RPS,
                                              N - (NS - 1) * RPS)])


_SC_KERNEL_CACHE = []


def _sc_gather_scatter(h, src, dst, w):
    if not _SC_KERNEL_CACHE:
        _SC_KERNEL_CACHE.append(pl.kernel(
            _sc_body,
            out_type=jax.ShapeDtypeStruct((NC, N, H), jnp.float32),
            mesh=plsc.VectorSubcoreMesh(core_axis_name="c",
                                        subcore_axis_name="s",
                                        num_cores=NC, num_subcores=NS),
            scratch_types=[
                pltpu.VMEM_SHARED((APAD, H), jnp.float32),
                pltpu.VMEM((1, K), jnp.int32),
                pltpu.VMEM((1, K), jnp.int32),
                pltpu.VMEM((1, K), jnp.int32),
                pltpu.VMEM((1, K), jnp.int32),
                pltpu.VMEM((1, K), jnp.int32),
                pltpu.VMEM((1, K), jnp.int32),
                pltpu.VMEM((1, K), jnp.float32),
                pltpu.VMEM((1, K), jnp.float32),
                pltpu.VMEM((K, H), jnp.float32),
                pltpu.VMEM((K, H), jnp.float32),
                pltpu.SemaphoreType.DMA,
                pltpu.SemaphoreType.DMA,
                pltpu.SemaphoreType.DMA,
                pltpu.SemaphoreType.DMA,
                pltpu.SemaphoreType.DMA,
                pltpu.SemaphoreType.DMA,
            ],
        ))
    return _SC_KERNEL_CACHE[0](h, src, dst, w)


def _lstm_body(x_ref, wih_ref, whh_ref, b_ref, h_ref):
    wih = wih_ref[...]
    whh = whh_ref[...]
    b = b_ref[...]
    h = jnp.zeros((BN, H), jnp.float32)
    c = jnp.zeros((BN, H), jnp.float32)
    for t in range(T):
        g = (jnp.dot(x_ref[:, t, :], wih, preferred_element_type=jnp.float32)
             + jnp.dot(h, whh, preferred_element_type=jnp.float32) + b)
        i = jax.nn.sigmoid(g[:, :H])
        f = jax.nn.sigmoid(g[:, H:2 * H])
        gg = jnp.tanh(g[:, 2 * H:3 * H])
        o = jax.nn.sigmoid(g[:, 3 * H:])
        c = f * c + i * gg
        h = o * jnp.tanh(c)
    h_ref[...] = h


def _lstm_call(xs, W_ih, W_hh, b2):
    return pl.pallas_call(
        _lstm_body,
        grid=(GRID,),
        in_specs=[
            pl.BlockSpec((BN, T, D), lambda i: (i, 0, 0)),
            pl.BlockSpec((D, 4 * H), lambda i: (0, 0)),
            pl.BlockSpec((H, 4 * H), lambda i: (0, 0)),
            pl.BlockSpec((1, 4 * H), lambda i: (0, 0)),
        ],
        out_specs=pl.BlockSpec((BN, H), lambda i: (i, 0)),
        out_shape=jax.ShapeDtypeStruct((N, H), jnp.float32),
    )(xs, W_ih, W_hh, b2)


def _edgew_body(k_ref, x_ref, o_ref):
    k = jax.nn.softplus(k_ref[...]) + 1e-3
    xr = 0.5 * jax.nn.sigmoid(x_ref[...])
    denom = k * (1.0 - xr) + 0.5 * DT
    o_ref[...] = (0.5 * DT - k * xr) / denom


def _edgew_call(kp, xp):
    return pl.pallas_call(
        _edgew_body,
        out_shape=jax.ShapeDtypeStruct((EW_R, 128), jnp.float32),
    )(kp, xp)


def _glayer_body(h_ref, p_ref, ws_ref, wn_ref, b_ref, o_ref):
    agg = p_ref[0] + p_ref[1]
    o_ref[...] = jnp.maximum(
        jnp.dot(h_ref[...], ws_ref[...], preferred_element_type=jnp.float32)
        + jnp.dot(agg, wn_ref[...], preferred_element_type=jnp.float32)
        + b_ref[...], 0.0)


def _glayer_call(h, p, Ws, Wn, b2):
    return pl.pallas_call(
        _glayer_body,
        grid=(GRID,),
        in_specs=[
            pl.BlockSpec((BN, H), lambda i: (i, 0)),
            pl.BlockSpec((NC, BN, H), lambda i: (0, i, 0)),
            pl.BlockSpec((H, H), lambda i: (0, 0)),
            pl.BlockSpec((H, H), lambda i: (0, 0)),
            pl.BlockSpec((1, H), lambda i: (0, 0)),
        ],
        out_specs=pl.BlockSpec((BN, H), lambda i: (i, 0)),
        out_shape=jax.ShapeDtypeStruct((N, H), jnp.float32),
    )(h, p, Ws, Wn, b2)


def _glayer_head_body(h_ref, p_ref, ws_ref, wn_ref, b_ref, wh_ref, bh_ref,
                      o_ref):
    agg = p_ref[0] + p_ref[1]
    h2 = jnp.maximum(
        jnp.dot(h_ref[...], ws_ref[...], preferred_element_type=jnp.float32)
        + jnp.dot(agg, wn_ref[...], preferred_element_type=jnp.float32)
        + b_ref[...], 0.0)
    o_ref[...] = (jnp.dot(h2, wh_ref[...], preferred_element_type=jnp.float32)
                  + bh_ref[...])


def _glayer_head_call(h, p, Ws, Wn, b2, Whp, bhp):
    return pl.pallas_call(
        _glayer_head_body,
        grid=(GRID,),
        in_specs=[
            pl.BlockSpec((BN, H), lambda i: (i, 0)),
            pl.BlockSpec((NC, BN, H), lambda i: (0, i, 0)),
            pl.BlockSpec((H, H), lambda i: (0, 0)),
            pl.BlockSpec((H, H), lambda i: (0, 0)),
            pl.BlockSpec((1, H), lambda i: (0, 0)),
            pl.BlockSpec((H, 128), lambda i: (0, 0)),
            pl.BlockSpec((1, 128), lambda i: (0, 0)),
        ],
        out_specs=pl.BlockSpec((BN, 128), lambda i: (i, 0)),
        out_shape=jax.ShapeDtypeStruct((N, 128), jnp.float32),
    )(h, p, Ws, Wn, b2, Whp, bhp)


def kernel(x, edge_index, W_ih, W_hh, b_lstm, k_param, x_param,
           W_self0, W_neigh0, b0, W_self1, W_neigh1, b1, W_head, b_head):
    xs = x[0]
    src = edge_index[0].reshape(NW, NCHUNK, 1, K)
    dst = edge_index[1].reshape(NW, NCHUNK, 1, K)
    w = _edgew_call(k_param.reshape(EW_R, 128),
                    x_param.reshape(EW_R, 128)).reshape(NW, NCHUNK, 1, K)

    h = _lstm_call(xs, W_ih, W_hh, b_lstm.reshape(1, 4 * H))

    p0 = _sc_gather_scatter(h, src, dst, w)
    h1 = _glayer_call(h, p0, W_self0, W_neigh0, b0.reshape(1, H))

    p1 = _sc_gather_scatter(h1, src, dst, w)
    Whp = jnp.zeros((H, 128), jnp.float32).at[:, :HOR].set(W_head)
    bhp = jnp.zeros((1, 128), jnp.float32).at[0, :HOR].set(b_head)
    out = _glayer_head_call(h1, p1, W_self1, W_neigh1, b1.reshape(1, H),
                            Whp, bhp)
    return out[:, :HOR][None]


# speedup vs baseline: 7.6741x; 1.0018x over previous
import jax
import jax.numpy as jnp
from jax import lax
from jax.experimental import pallas as pl
from jax.experimental.pallas import tpu as pltpu
from jax.experimental.pallas import tpu_sc as plsc

N = 10000
E = 320000
D = 128
H = 128
T = 12
HOR = 24
DT = 1.0

NC = 2
NS = 16
NW = NC * NS
EPW = E // NW
K = 80
NCHUNK = EPW // K
APAD = 10240
RPS = APAD // NS
ZROWS = 128
LN = 16

BN = 1000
GRID = N // BN

EW_R = E // 128


def _sc_body(h_hbm, src_hbm, dst_hbm, w_hbm, out_hbm, agg_sh,
             src0, src1, dst0, dst1, dsc0, dsc1, w0, w1, rows0, rows1,
             gsem0, gsem1, isem0, isem1, ssem0, ssem1):
    cid = lax.axis_index("c")
    sid = lax.axis_index("s")
    wid = sid * NC + cid

    srcb = (src0, src1)
    dstb = (dst0, dst1)
    dscb = (dsc0, dsc1)
    wb = (w0, w1)
    rowsb = (rows0, rows1)
    gsem = (gsem0, gsem1)
    isem = (isem0, isem1)
    ssem = (ssem0, ssem1)

    def stage(j, s):
        pltpu.async_copy(src_hbm.at[wid, j], srcb[s], isem[s])
        pltpu.async_copy(dst_hbm.at[wid, j], dstb[s], isem[s])
        pltpu.async_copy(w_hbm.at[wid, j], wb[s], isem[s])

    def wait_stage(s):
        pltpu.make_async_copy(src_hbm.at[0, 0], srcb[s], isem[s]).wait()
        pltpu.make_async_copy(dst_hbm.at[0, 0], dstb[s], isem[s]).wait()
        pltpu.make_async_copy(w_hbm.at[0, 0], wb[s], isem[s]).wait()

    def gissue(s):
        pltpu.async_copy(h_hbm.at[srcb[s].at[0]], rowsb[s], gsem[s])

    def gwait(s):
        pltpu.make_async_copy(h_hbm.at[srcb[s].at[0]], rowsb[s],
                              gsem[s]).wait()

    def scatter_issue(s):
        pltpu.async_copy(rowsb[s], agg_sh.at[dscb[s].at[0]], ssem[s],
                         add=True)

    def wait_scatter(s):
        pltpu.make_async_copy(rowsb[s], agg_sh.at[dscb[s].at[0]],
                              ssem[s]).wait()

    def process(s):
        def group(g, c2):
            wv = wb[s][0, pl.ds(g * LN, LN)]
            base = g * LN
            for l in range(LN):
                ws = wv[l]
                for q in range(H // LN):
                    rowsb[s][base + l, pl.ds(q * LN, LN)] = (
                        rowsb[s][base + l, pl.ds(q * LN, LN)] * ws)
            return c2

        lax.fori_loop(0, K // LN, group, 0)
        for g in range(K // LN):
            dscb[s][0, pl.ds(g * LN, LN)] = dstb[s][0, pl.ds(g * LN, LN)]
        scatter_issue(s)

    stage(0, 0)
    stage(1, 1)
    zv = jnp.zeros((LN,), jnp.float32)

    def zrow(i, carry):
        for q in range(H // LN):
            rows1[i, pl.ds(q * LN, LN)] = zv
        return carry

    lax.fori_loop(0, K, zrow, 0)
    for r in range(RPS // K):
        pltpu.sync_copy(rows1, agg_sh.at[pl.ds(sid * RPS + r * K, K)])
    wait_stage(0)
    gissue(0)
    plsc.subcore_barrier()

    def half(j, s):
        @pl.when((j + 1 < NCHUNK) & (j >= 1))
        def _drain():
            wait_scatter(1 - s)

        @pl.when(j + 1 < NCHUNK)
        def _prefetch():
            wait_stage(1 - s)
            gissue(1 - s)

        @pl.when(j < NCHUNK)
        def _proc():
            gwait(s)
            process(s)

        @pl.when(j + 2 < NCHUNK)
        def _restage():
            stage(j + 2, s)

    def pair(jj, carry):
        half(2 * jj, 0)
        half(2 * jj + 1, 1)
        return carry

    lax.fori_loop(0, (NCHUNK + 1) // 2, pair, 0)
    wait_scatter(0)
    wait_scatter(1)
    plsc.subcore_barrier()

    @pl.when(sid < NS - 1)
    def _full():
        pltpu.sync_copy(agg_sh.at[pl.ds(sid * RPS, RPS)],
                        out_hbm.at[cid, pl.ds(sid * RPS, RPS)])

    @pl.when(sid == NS - 1)
    def _tail():
        pltpu.sync_copy(agg_sh.at[pl.ds((NS - 1) * RPS, N - (NS - 1) * RPS)],
                        out_hbm.at[cid, pl.ds((NS - 1) * RPS,
                                              N - (NS - 1) * RPS)])


_SC_KERNEL_CACHE = []


def _sc_gather_scatter(h, src, dst, w):
    if not _SC_KERNEL_CACHE:
        _SC_KERNEL_CACHE.append(pl.kernel(
            _sc_body,
            out_type=jax.ShapeDtypeStruct((NC, N, H), jnp.float32),
            mesh=plsc.VectorSubcoreMesh(core_axis_name="c",
                                        subcore_axis_name="s",
                                        num_cores=NC, num_subcores=NS),
            compiler_params=pltpu.CompilerParams(use_tc_tiling_on_sc=True),
            scratch_types=[
                pltpu.VMEM_SHARED((APAD, H), jnp.float32),
                pltpu.VMEM((1, K), jnp.int32),
                pltpu.VMEM((1, K), jnp.int32),
                pltpu.VMEM((1, K), jnp.int32),
                pltpu.VMEM((1, K), jnp.int32),
                pltpu.VMEM((1, K), jnp.int32),
                pltpu.VMEM((1, K), jnp.int32),
                pltpu.VMEM((1, K), jnp.float32),
                pltpu.VMEM((1, K), jnp.float32),
                pltpu.VMEM((K, H), jnp.float32),
                pltpu.VMEM((K, H), jnp.float32),
                pltpu.SemaphoreType.DMA,
                pltpu.SemaphoreType.DMA,
                pltpu.SemaphoreType.DMA,
                pltpu.SemaphoreType.DMA,
                pltpu.SemaphoreType.DMA,
                pltpu.SemaphoreType.DMA,
            ],
        ))
    return _SC_KERNEL_CACHE[0](h, src, dst, w)


def _lstm_body(x_ref, wih_ref, whh_ref, b_ref, h_ref):
    wih = wih_ref[...]
    whh = whh_ref[...]
    b = b_ref[...]
    h = jnp.zeros((BN, H), jnp.float32)
    c = jnp.zeros((BN, H), jnp.float32)
    for t in range(T):
        g = (jnp.dot(x_ref[:, t, :], wih, preferred_element_type=jnp.float32)
             + jnp.dot(h, whh, preferred_element_type=jnp.float32) + b)
        i = jax.nn.sigmoid(g[:, :H])
        f = jax.nn.sigmoid(g[:, H:2 * H])
        gg = jnp.tanh(g[:, 2 * H:3 * H])
        o = jax.nn.sigmoid(g[:, 3 * H:])
        c = f * c + i * gg
        h = o * jnp.tanh(c)
    h_ref[...] = h


def _lstm_call(xs, W_ih, W_hh, b2):
    return pl.pallas_call(
        _lstm_body,
        grid=(GRID,),
        in_specs=[
            pl.BlockSpec((BN, T, D), lambda i: (i, 0, 0)),
            pl.BlockSpec((D, 4 * H), lambda i: (0, 0)),
            pl.BlockSpec((H, 4 * H), lambda i: (0, 0)),
            pl.BlockSpec((1, 4 * H), lambda i: (0, 0)),
        ],
        out_specs=pl.BlockSpec((BN, H), lambda i: (i, 0)),
        out_shape=jax.ShapeDtypeStruct((N, H), jnp.float32),
    )(xs, W_ih, W_hh, b2)


def _edgew_body(k_ref, x_ref, o_ref):
    k = jax.nn.softplus(k_ref[...]) + 1e-3
    xr = 0.5 * jax.nn.sigmoid(x_ref[...])
    denom = k * (1.0 - xr) + 0.5 * DT
    o_ref[...] = (0.5 * DT - k * xr) / denom


def _edgew_call(kp, xp):
    return pl.pallas_call(
        _edgew_body,
        out_shape=jax.ShapeDtypeStruct((EW_R, 128), jnp.float32),
    )(kp, xp)


def _glayer_body(h_ref, p_ref, ws_ref, wn_ref, b_ref, o_ref):
    agg = p_ref[0] + p_ref[1]
    o_ref[...] = jnp.maximum(
        jnp.dot(h_ref[...], ws_ref[...], preferred_element_type=jnp.float32)
        + jnp.dot(agg, wn_ref[...], preferred_element_type=jnp.float32)
        + b_ref[...], 0.0)


def _glayer_call(h, p, Ws, Wn, b2):
    return pl.pallas_call(
        _glayer_body,
        grid=(GRID,),
        in_specs=[
            pl.BlockSpec((BN, H), lambda i: (i, 0)),
            pl.BlockSpec((NC, BN, H), lambda i: (0, i, 0)),
            pl.BlockSpec((H, H), lambda i: (0, 0)),
            pl.BlockSpec((H, H), lambda i: (0, 0)),
            pl.BlockSpec((1, H), lambda i: (0, 0)),
        ],
        out_specs=pl.BlockSpec((BN, H), lambda i: (i, 0)),
        out_shape=jax.ShapeDtypeStruct((N, H), jnp.float32),
    )(h, p, Ws, Wn, b2)


def _glayer_head_body(h_ref, p_ref, ws_ref, wn_ref, b_ref, wh_ref, bh_ref,
                      o_ref):
    agg = p_ref[0] + p_ref[1]
    h2 = jnp.maximum(
        jnp.dot(h_ref[...], ws_ref[...], preferred_element_type=jnp.float32)
        + jnp.dot(agg, wn_ref[...], preferred_element_type=jnp.float32)
        + b_ref[...], 0.0)
    o_ref[...] = (jnp.dot(h2, wh_ref[...], preferred_element_type=jnp.float32)
                  + bh_ref[...])


def _glayer_head_call(h, p, Ws, Wn, b2, Whp, bhp):
    return pl.pallas_call(
        _glayer_head_body,
        grid=(GRID,),
        in_specs=[
            pl.BlockSpec((BN, H), lambda i: (i, 0)),
            pl.BlockSpec((NC, BN, H), lambda i: (0, i, 0)),
            pl.BlockSpec((H, H), lambda i: (0, 0)),
            pl.BlockSpec((H, H), lambda i: (0, 0)),
            pl.BlockSpec((1, H), lambda i: (0, 0)),
            pl.BlockSpec((H, 128), lambda i: (0, 0)),
            pl.BlockSpec((1, 128), lambda i: (0, 0)),
        ],
        out_specs=pl.BlockSpec((BN, 128), lambda i: (i, 0)),
        out_shape=jax.ShapeDtypeStruct((N, 128), jnp.float32),
    )(h, p, Ws, Wn, b2, Whp, bhp)


def kernel(x, edge_index, W_ih, W_hh, b_lstm, k_param, x_param,
           W_self0, W_neigh0, b0, W_self1, W_neigh1, b1, W_head, b_head):
    xs = x[0]
    src = edge_index[0].reshape(NW, NCHUNK, 1, K)
    dst = edge_index[1].reshape(NW, NCHUNK, 1, K)
    w = _edgew_call(k_param.reshape(EW_R, 128),
                    x_param.reshape(EW_R, 128)).reshape(NW, NCHUNK, 1, K)

    h = _lstm_call(xs, W_ih, W_hh, b_lstm.reshape(1, 4 * H))

    p0 = _sc_gather_scatter(h, src, dst, w)
    h1 = _glayer_call(h, p0, W_self0, W_neigh0, b0.reshape(1, H))

    p1 = _sc_gather_scatter(h1, src, dst, w)
    Whp = jnp.zeros((H, 128), jnp.float32).at[:, :HOR].set(W_head)
    bhp = jnp.zeros((1, 128), jnp.float32).at[0, :HOR].set(b_head)
    out = _glayer_head_call(h1, p1, W_self1, W_neigh1, b1.reshape(1, H),
                            Whp, bhp)
    return out[:, :HOR][None]
